# baseline (speedup 1.0000x reference)
import jax
import jax.numpy as jnp
from jax.experimental import pallas as pl
from jax.experimental.pallas import tpu as pltpu

_B, _K, _C, _W, _H = 256, 133, 256, 768, 1024
_KTOP = 10


def _body(hx_ref, hy_ref, f_ref, canon_ref, out_f_ref, out_m_ref):
    hx = hx_ref[0]
    hy = hy_ref[0]
    mx = jnp.max(hx, axis=-1, keepdims=True)
    sx = jnp.sum(jnp.exp(hx - mx), axis=-1, keepdims=True)
    my = jnp.max(hy, axis=-1, keepdims=True)
    sy = jnp.sum(jnp.exp(hy - my), axis=-1, keepdims=True)
    score = 1.0 - 0.5 * (1.0 / sx + 1.0 / sy)

    iota = jax.lax.broadcasted_iota(jnp.int32, (_K, 1), 0)
    mask = jnp.zeros((_K, 1), dtype=jnp.bool_)
    for _ in range(_KTOP):
        cur = jnp.where(mask, -1.0, score)
        m = jnp.max(cur)
        idx = jnp.min(jnp.where(cur == m, iota, _K))
        mask = mask | (iota == idx)

    out_m_ref[0] = mask.astype(jnp.int32)
    out_f_ref[0] = jnp.where(mask, canon_ref[...], f_ref[0])


def kernel(f_kpts, h_initial_x, h_initial_y, canonical_table):
    out_f, out_m = pl.pallas_call(
        _body,
        grid=(_B,),
        in_specs=[
            pl.BlockSpec((1, _K, _W), lambda b: (b, 0, 0)),
            pl.BlockSpec((1, _K, _H), lambda b: (b, 0, 0)),
            pl.BlockSpec((1, _K, _C), lambda b: (b, 0, 0)),
            pl.BlockSpec((_K, _C), lambda b: (0, 0)),
        ],
        out_specs=[
            pl.BlockSpec((1, _K, _C), lambda b: (b, 0, 0)),
            pl.BlockSpec((1, _K, 1), lambda b: (b, 0, 0)),
        ],
        out_shape=[
            jax.ShapeDtypeStruct((_B, _K, _C), jnp.float32),
            jax.ShapeDtypeStruct((_B, _K, 1), jnp.int32),
        ],
        compiler_params=pltpu.CompilerParams(
            dimension_semantics=("parallel",),
        ),
    )(h_initial_x, h_initial_y, f_kpts, canonical_table)
    return out_f, (out_m[:, :, 0] != 0)


# speedup vs baseline: 1.5766x; 1.5766x over previous
import jax
import jax.numpy as jnp
from jax.experimental import pallas as pl
from jax.experimental.pallas import tpu as pltpu

_B, _K, _C, _W, _H = 256, 133, 256, 768, 1024
_KTOP = 10
_TB = 8
_NB = _B // _TB


def _scores_body(hx_ref, hy_ref, out_ref):
    cols = []
    for tb in range(_TB):
        hx = hx_ref[tb]
        hy = hy_ref[tb]
        sx = jnp.sum(jnp.exp(hx - jnp.max(hx, axis=-1, keepdims=True)),
                     axis=-1, keepdims=True)
        sy = jnp.sum(jnp.exp(hy - jnp.max(hy, axis=-1, keepdims=True)),
                     axis=-1, keepdims=True)
        cols.append(1.0 - 0.5 * (1.0 / sx + 1.0 / sy))
    out_ref[0] = jnp.concatenate(cols, axis=1)


def _topk_body(s_ref, m_ref):
    score = s_ref[...]
    iota = jax.lax.broadcasted_iota(jnp.int32, (_NB, _K, _TB), 1)
    mask = jnp.zeros((_NB, _K, _TB), dtype=jnp.bool_)
    for _ in range(_KTOP):
        cur = jnp.where(mask, -1.0, score)
        m = jnp.max(cur, axis=1, keepdims=True)
        idx = jnp.min(jnp.where(cur == m, iota, _K),
                      axis=1, keepdims=True)
        mask = mask | (iota == idx)
    m_ref[...] = mask.astype(jnp.int32)


def _select_body(f_ref, m_ref, canon_ref, out_ref):
    canon = canon_ref[...]
    for tb in range(_TB):
        msk = m_ref[0][:, tb:tb + 1] != 0
        out_ref[tb] = jnp.where(msk, canon, f_ref[tb])


def kernel(f_kpts, h_initial_x, h_initial_y, canonical_table):
    scores_t = pl.pallas_call(
        _scores_body,
        grid=(_NB,),
        in_specs=[
            pl.BlockSpec((_TB, _K, _W), lambda i: (i, 0, 0)),
            pl.BlockSpec((_TB, _K, _H), lambda i: (i, 0, 0)),
        ],
        out_specs=pl.BlockSpec((1, _K, _TB), lambda i: (i, 0, 0)),
        out_shape=jax.ShapeDtypeStruct((_NB, _K, _TB), jnp.float32),
        compiler_params=pltpu.CompilerParams(
            dimension_semantics=("parallel",),
        ),
    )(h_initial_x, h_initial_y)

    mask_t = pl.pallas_call(
        _topk_body,
        out_shape=jax.ShapeDtypeStruct((_NB, _K, _TB), jnp.int32),
    )(scores_t)

    out_f = pl.pallas_call(
        _select_body,
        grid=(_NB,),
        in_specs=[
            pl.BlockSpec((_TB, _K, _C), lambda i: (i, 0, 0)),
            pl.BlockSpec((1, _K, _TB), lambda i: (i, 0, 0)),
            pl.BlockSpec((_K, _C), lambda i: (0, 0)),
        ],
        out_specs=pl.BlockSpec((_TB, _K, _C), lambda i: (i, 0, 0)),
        out_shape=jax.ShapeDtypeStruct((_B, _K, _C), jnp.float32),
        compiler_params=pltpu.CompilerParams(
            dimension_semantics=("parallel",),
        ),
    )(f_kpts, mask_t, canonical_table)

    mask = jnp.transpose(mask_t, (0, 2, 1)).reshape(_B, _K) != 0
    return out_f, mask


# speedup vs baseline: 1.6805x; 1.0659x over previous
import jax
import jax.numpy as jnp
from jax.experimental import pallas as pl
from jax.experimental.pallas import tpu as pltpu

_B, _K, _C, _W, _H = 256, 133, 256, 768, 1024
_KTOP = 10
_TB = 8
_NB = _B // _TB


def _body(hx_ref, hy_ref, f_ref, canon_ref, out_f_ref, out_m_ref):
    cols = []
    for tb in range(_TB):
        hx = hx_ref[tb]
        hy = hy_ref[tb]
        sx = jnp.sum(jnp.exp(hx - jnp.max(hx, axis=-1, keepdims=True)),
                     axis=-1, keepdims=True)
        sy = jnp.sum(jnp.exp(hy - jnp.max(hy, axis=-1, keepdims=True)),
                     axis=-1, keepdims=True)
        cols.append(1.0 - 0.5 * (1.0 / sx + 1.0 / sy))
    score = jnp.concatenate(cols, axis=1)

    iota = jax.lax.broadcasted_iota(jnp.int32, (_K, _TB), 0)
    mask = jnp.zeros((_K, _TB), dtype=jnp.bool_)
    for _ in range(_KTOP):
        cur = jnp.where(mask, -1.0, score)
        m = jnp.max(cur, axis=0, keepdims=True)
        idx = jnp.min(jnp.where(cur == m, iota, _K),
                      axis=0, keepdims=True)
        mask = mask | (iota == idx)
    out_m_ref[0] = mask.astype(jnp.int32)

    canon = canon_ref[...]
    for tb in range(_TB):
        out_f_ref[tb] = jnp.where(mask[:, tb:tb + 1], canon, f_ref[tb])


def kernel(f_kpts, h_initial_x, h_initial_y, canonical_table):
    out_f, mask_t = pl.pallas_call(
        _body,
        grid=(_NB,),
        in_specs=[
            pl.BlockSpec((_TB, _K, _W), lambda i: (i, 0, 0)),
            pl.BlockSpec((_TB, _K, _H), lambda i: (i, 0, 0)),
            pl.BlockSpec((_TB, _K, _C), lambda i: (i, 0, 0)),
            pl.BlockSpec((_K, _C), lambda i: (0, 0)),
        ],
        out_specs=[
            pl.BlockSpec((_TB, _K, _C), lambda i: (i, 0, 0)),
            pl.BlockSpec((1, _K, _TB), lambda i: (i, 0, 0)),
        ],
        out_shape=[
            jax.ShapeDtypeStruct((_B, _K, _C), jnp.float32),
            jax.ShapeDtypeStruct((_NB, _K, _TB), jnp.int32),
        ],
        compiler_params=pltpu.CompilerParams(
            dimension_semantics=("parallel",),
        ),
    )(h_initial_x, h_initial_y, f_kpts, canonical_table)

    mask = jnp.transpose(mask_t, (0, 2, 1)).reshape(_B, _K) != 0
    return out_f, mask


# speedup vs baseline: 1.7111x; 1.0182x over previous
import jax
import jax.numpy as jnp
from jax.experimental import pallas as pl
from jax.experimental.pallas import tpu as pltpu

_B, _K, _C, _W, _H = 256, 133, 256, 768, 1024
_KTOP = 10
_TB = 16
_NB = _B // _TB


def _body(hx_ref, hy_ref, f_ref, canon_ref, out_f_ref, out_m_ref):
    cols = []
    for tb in range(_TB):
        hx = hx_ref[tb]
        hy = hy_ref[tb]
        sx = jnp.sum(jnp.exp(hx - jnp.max(hx, axis=-1, keepdims=True)),
                     axis=-1, keepdims=True)
        sy = jnp.sum(jnp.exp(hy - jnp.max(hy, axis=-1, keepdims=True)),
                     axis=-1, keepdims=True)
        cols.append(1.0 - 0.5 * (1.0 / sx + 1.0 / sy))
    score = jnp.concatenate(cols, axis=1)

    iota = jax.lax.broadcasted_iota(jnp.int32, (_K, _TB), 0)
    mask = jnp.zeros((_K, _TB), dtype=jnp.bool_)
    for _ in range(_KTOP):
        cur = jnp.where(mask, -1.0, score)
        m = jnp.max(cur, axis=0, keepdims=True)
        idx = jnp.min(jnp.where(cur == m, iota, _K),
                      axis=0, keepdims=True)
        mask = mask | (iota == idx)
    out_m_ref[0] = mask.astype(jnp.int32)

    canon = canon_ref[...]
    for tb in range(_TB):
        out_f_ref[tb] = jnp.where(mask[:, tb:tb + 1], canon, f_ref[tb])


def kernel(f_kpts, h_initial_x, h_initial_y, canonical_table):
    out_f, mask_t = pl.pallas_call(
        _body,
        grid=(_NB,),
        in_specs=[
            pl.BlockSpec((_TB, _K, _W), lambda i: (i, 0, 0)),
            pl.BlockSpec((_TB, _K, _H), lambda i: (i, 0, 0)),
            pl.BlockSpec((_TB, _K, _C), lambda i: (i, 0, 0)),
            pl.BlockSpec((_K, _C), lambda i: (0, 0)),
        ],
        out_specs=[
            pl.BlockSpec((_TB, _K, _C), lambda i: (i, 0, 0)),
            pl.BlockSpec((1, _K, _TB), lambda i: (i, 0, 0)),
        ],
        out_shape=[
            jax.ShapeDtypeStruct((_B, _K, _C), jnp.float32),
            jax.ShapeDtypeStruct((_NB, _K, _TB), jnp.int32),
        ],
        compiler_params=pltpu.CompilerParams(
            dimension_semantics=("parallel",),
        ),
    )(h_initial_x, h_initial_y, f_kpts, canonical_table)

    mask = jnp.transpose(mask_t, (0, 2, 1)).reshape(_B, _K) != 0
    return out_f, mask
